# baseline (speedup 1.0000x reference)
import jax
import jax.numpy as jnp
from jax import lax
from jax.experimental import pallas as pl
from jax.experimental.pallas import tpu as pltpu

_H = 1024
_NC = 81
_BN_EPS = 1e-3


def _bn_relu(h, gamma, beta):
    mean = jnp.mean(h, axis=0, keepdims=True)
    var = jnp.mean((h - mean) * (h - mean), axis=0, keepdims=True)
    inv = lax.rsqrt(var + _BN_EPS)
    return jnp.maximum((h - mean) * inv * gamma + beta, 0.0)


def _body(x_hbm, w1_hbm, w2_hbm, b1_ref, g1_ref, be1_ref, b2_ref, g2_ref,
          be2_ref, lw_ref, lb_ref, dw_ref, db_ref,
          logits_ref, probs_ref, deltas_ref,
          acc_ref, xr0, xr1, ws0, ws1, ws2, ws3, w2b, semx, semw, semw2):
    n = xr0.shape[0]
    quarter = n // 4
    xr = (xr0, xr1)
    ws = (ws0, ws1, ws2, ws3)

    def xcps(h, s):
        return [pltpu.make_async_copy(
                    x_hbm.at[pl.ds(c * quarter, quarter), h, :, :],
                    xr[s].at[pl.ds(c * quarter, quarter), :, :],
                    semx.at[s]) for c in range(4)]

    def wcp(u, s):
        h, c = u // 4, u % 4
        return pltpu.make_async_copy(
            w1_hbm.at[h, :, :, pl.ds(c * 256, 256)],
            ws[s], semw.at[s])

    def start_xrow(h, s):
        for cp in xcps(h, s):
            cp.start()

    start_xrow(0, 0)
    start_xrow(1, 1)
    for u0 in range(4):
        wcp(u0, u0).start()
    pltpu.make_async_copy(w2_hbm.at[0, 0], w2b, semw2).start()

    for h in range(7):
        s = h % 2
        for cp in xcps(h, s):
            cp.wait()
        xb = xr[s][...].reshape(n, 1792).astype(jnp.bfloat16)
        for c in range(4):
            u = 4 * h + c
            su = u % 4
            wcp(u, su).wait()
            wb = ws[su][...].reshape(1792, 256).astype(jnp.bfloat16)
            d = jnp.dot(xb, wb, preferred_element_type=jnp.float32)
            if h == 0:
                acc_ref[:, pl.ds(c * 256, 256)] = d
            else:
                acc_ref[:, pl.ds(c * 256, 256)] += d
            if u + 4 < 28:
                wcp(u + 4, su).start()
        if h + 2 < 7:
            start_xrow(h + 2, s)

    h1 = acc_ref[...] + b1_ref[...]
    x1 = _bn_relu(h1, g1_ref[...], be1_ref[...])
    pltpu.make_async_copy(w2_hbm.at[0, 0], w2b, semw2).wait()
    h2 = jnp.dot(x1.astype(jnp.bfloat16), w2b[...].astype(jnp.bfloat16),
                 preferred_element_type=jnp.float32)
    h2 = h2 + b2_ref[...]
    x2 = _bn_relu(h2, g2_ref[...], be2_ref[...])
    logits = jnp.dot(x2, lw_ref[...], preferred_element_type=jnp.float32)
    logits = logits + lb_ref[...]
    logits_ref[...] = logits
    m = jnp.max(logits, axis=-1, keepdims=True)
    e = jnp.exp(logits - m)
    probs_ref[...] = e / jnp.sum(e, axis=-1, keepdims=True)
    d = jnp.dot(x2, dw_ref[...], preferred_element_type=jnp.float32)
    deltas_ref[...] = d + db_ref[...]


def kernel(pooled_rois, conv1_w, conv1_b, bn1_gamma, bn1_beta, conv2_w,
           conv2_b, bn2_gamma, bn2_beta, logits_w, logits_b, delta_w,
           delta_b):
    n = pooled_rois.shape[0]
    row = lambda v: v.reshape(1, -1)

    logits, probs, deltas = pl.pallas_call(
        _body,
        in_specs=[pl.BlockSpec(memory_space=pl.ANY)] * 3
                 + [pl.BlockSpec()] * 10,
        out_specs=[pl.BlockSpec()] * 3,
        out_shape=[
            jax.ShapeDtypeStruct((n, _NC), jnp.float32),
            jax.ShapeDtypeStruct((n, _NC), jnp.float32),
            jax.ShapeDtypeStruct((n, 4 * _NC), jnp.float32),
        ],
        scratch_shapes=[
            pltpu.VMEM((n, _H), jnp.float32),
            pltpu.VMEM((n, 7, 256), jnp.float32),
            pltpu.VMEM((n, 7, 256), jnp.float32),
            pltpu.VMEM((7, 256, 256), jnp.float32),
            pltpu.VMEM((7, 256, 256), jnp.float32),
            pltpu.VMEM((7, 256, 256), jnp.float32),
            pltpu.VMEM((7, 256, 256), jnp.float32),
            pltpu.VMEM((_H, _H), jnp.float32),
            pltpu.SemaphoreType.DMA((2,)),
            pltpu.SemaphoreType.DMA((4,)),
            pltpu.SemaphoreType.DMA,
        ],
        compiler_params=pltpu.CompilerParams(
            vmem_limit_bytes=100 * 1024 * 1024,
        ),
    )(pooled_rois, conv1_w, conv2_w, row(conv1_b), row(bn1_gamma),
      row(bn1_beta), row(conv2_b), row(bn2_gamma), row(bn2_beta), logits_w,
      row(logits_b), delta_w, row(delta_b))
    return logits, probs, deltas.reshape(n, _NC, 4)


# speedup vs baseline: 1.2984x; 1.2984x over previous
import jax
import jax.numpy as jnp
from jax import lax
from jax.experimental import pallas as pl
from jax.experimental.pallas import tpu as pltpu

_H = 1024
_NC = 81
_BN_EPS = 1e-3


def _bn_relu(h, gamma, beta):
    mean = jnp.mean(h, axis=0, keepdims=True)
    var = jnp.mean((h - mean) * (h - mean), axis=0, keepdims=True)
    inv = lax.rsqrt(var + _BN_EPS)
    return jnp.maximum((h - mean) * inv * gamma + beta, 0.0)


def _body(xa_ref, xb_ref, wa_ref, wb_ref, b1_ref, g1_ref, be1_ref, w2_ref,
          b2_ref, g2_ref, be2_ref, lw_ref, lb_ref, dw_ref, db_ref,
          logits_ref, probs_ref, deltas_ref, acc_ref):
    step = pl.program_id(0)
    half = xa_ref.shape[0]

    wua = wa_ref[...].reshape(1792, 512).astype(jnp.bfloat16)
    wub = wb_ref[...].reshape(1792, 512).astype(jnp.bfloat16)
    wb16 = jnp.concatenate([wua, wub], axis=1)
    for c, xref in enumerate((xa_ref, xb_ref)):
        xb16 = xref[...].reshape(half, 1792).astype(jnp.bfloat16)
        d = jnp.dot(xb16, wb16, preferred_element_type=jnp.float32)
        sl = pl.ds(c * half, half)

        @pl.when(step == 0)
        def _():
            acc_ref[sl, :] = d

        @pl.when(step != 0)
        def _():
            acc_ref[sl, :] += d

    @pl.when(step == 6)
    def _():
        h1 = acc_ref[...] + b1_ref[...]
        x1 = _bn_relu(h1, g1_ref[...], be1_ref[...])
        h2 = jnp.dot(x1.astype(jnp.bfloat16),
                     w2_ref[0, 0].astype(jnp.bfloat16),
                     preferred_element_type=jnp.float32)
        h2 = h2 + b2_ref[...]
        x2 = _bn_relu(h2, g2_ref[...], be2_ref[...])
        logits = jnp.dot(x2, lw_ref[...], preferred_element_type=jnp.float32)
        logits = logits + lb_ref[...]
        logits_ref[...] = logits
        m = jnp.max(logits, axis=-1, keepdims=True)
        e = jnp.exp(logits - m)
        probs_ref[...] = e / jnp.sum(e, axis=-1, keepdims=True)
        d = jnp.dot(x2, dw_ref[...], preferred_element_type=jnp.float32)
        deltas_ref[...] = d + db_ref[...]


def kernel(pooled_rois, conv1_w, conv1_b, bn1_gamma, bn1_beta, conv2_w,
           conv2_b, bn2_gamma, bn2_beta, logits_w, logits_b, delta_w,
           delta_b):
    n = pooled_rois.shape[0]
    half = n // 2
    row = lambda v: v.reshape(1, -1)

    full = lambda shape: pl.BlockSpec(shape, lambda s: (0,) * len(shape))
    logits, probs, deltas = pl.pallas_call(
        _body,
        grid=(7,),
        in_specs=[
            pl.BlockSpec((half, 1, 7, 256), lambda s: (0, s, 0, 0)),
            pl.BlockSpec((half, 1, 7, 256), lambda s: (1, s, 0, 0)),
            pl.BlockSpec((1, 7, 256, 512), lambda s: (s, 0, 0, 0)),
            pl.BlockSpec((1, 7, 256, 512), lambda s: (s, 0, 0, 1)),
            full((1, _H)), full((1, _H)), full((1, _H)),
            pl.BlockSpec((1, 1, _H, _H), lambda s: (0, 0, 0, 0)),
            full((1, _H)), full((1, _H)), full((1, _H)),
            full((_H, _NC)), full((1, _NC)),
            full((_H, 4 * _NC)), full((1, 4 * _NC)),
        ],
        out_specs=[
            full((n, _NC)),
            full((n, _NC)),
            full((n, 4 * _NC)),
        ],
        out_shape=[
            jax.ShapeDtypeStruct((n, _NC), jnp.float32),
            jax.ShapeDtypeStruct((n, _NC), jnp.float32),
            jax.ShapeDtypeStruct((n, 4 * _NC), jnp.float32),
        ],
        scratch_shapes=[
            pltpu.VMEM((n, _H), jnp.float32),
        ],
        compiler_params=pltpu.CompilerParams(
            dimension_semantics=("arbitrary",),
            vmem_limit_bytes=100 * 1024 * 1024,
        ),
    )(pooled_rois, pooled_rois, conv1_w, conv1_w, row(conv1_b),
      row(bn1_gamma), row(bn1_beta), conv2_w, row(conv2_b), row(bn2_gamma),
      row(bn2_beta), logits_w, row(logits_b), delta_w, row(delta_b))
    return logits, probs, deltas.reshape(n, _NC, 4)


# speedup vs baseline: 1.3381x; 1.0306x over previous
import jax
import jax.numpy as jnp
from jax import lax
from jax.experimental import pallas as pl
from jax.experimental.pallas import tpu as pltpu

_H = 1024
_NC = 81
_BN_EPS = 1e-3


def _bn_relu(h, gamma, beta):
    mean = jnp.mean(h, axis=0, keepdims=True)
    var = jnp.mean((h - mean) * (h - mean), axis=0, keepdims=True)
    inv = lax.rsqrt(var + _BN_EPS)
    return jnp.maximum((h - mean) * inv * gamma + beta, 0.0)


def _body(xa_ref, wa_ref, b1_ref, g1_ref, be1_ref, w2_ref,
          b2_ref, g2_ref, be2_ref, lw_ref, lb_ref, dw_ref, db_ref,
          logits_ref, probs_ref, deltas_ref, acc_ref):
    step = pl.program_id(0)
    half = xa_ref.shape[0]
    _H = 1024

    wb16 = wa_ref[...].reshape(1792, _H).astype(jnp.bfloat16)
    xb16 = xa_ref[...].reshape(half, 1792).astype(jnp.bfloat16)
    d = jnp.dot(xb16, wb16, preferred_element_type=jnp.float32)

    @pl.when(step == 0)
    def _():
        acc_ref[...] = d

    @pl.when(step != 0)
    def _():
        acc_ref[...] += d

    @pl.when(step == 6)
    def _():
        h1 = acc_ref[...] + b1_ref[...]
        x1 = _bn_relu(h1, g1_ref[...], be1_ref[...])
        h2 = jnp.dot(x1.astype(jnp.bfloat16),
                     w2_ref[0, 0].astype(jnp.bfloat16),
                     preferred_element_type=jnp.float32)
        h2 = h2 + b2_ref[...]
        x2 = _bn_relu(h2, g2_ref[...], be2_ref[...])
        logits = jnp.dot(x2, lw_ref[...], preferred_element_type=jnp.float32)
        logits = logits + lb_ref[...]
        logits_ref[...] = logits
        m = jnp.max(logits, axis=-1, keepdims=True)
        e = jnp.exp(logits - m)
        probs_ref[...] = e / jnp.sum(e, axis=-1, keepdims=True)
        d = jnp.dot(x2, dw_ref[...], preferred_element_type=jnp.float32)
        deltas_ref[...] = d + db_ref[...]


def kernel(pooled_rois, conv1_w, conv1_b, bn1_gamma, bn1_beta, conv2_w,
           conv2_b, bn2_gamma, bn2_beta, logits_w, logits_b, delta_w,
           delta_b):
    n = pooled_rois.shape[0]
    half = n // 2
    row = lambda v: v.reshape(1, -1)

    full = lambda shape: pl.BlockSpec(shape, lambda s: (0,) * len(shape))
    logits, probs, deltas = pl.pallas_call(
        _body,
        grid=(7,),
        in_specs=[
            pl.BlockSpec((n, 1, 7, 256), lambda s: (0, s, 0, 0)),
            pl.BlockSpec((1, 7, 256, _H), lambda s: (s, 0, 0, 0)),
            full((1, _H)), full((1, _H)), full((1, _H)),
            pl.BlockSpec((1, 1, _H, _H), lambda s: (0, 0, 0, 0)),
            full((1, _H)), full((1, _H)), full((1, _H)),
            full((_H, _NC)), full((1, _NC)),
            full((_H, 4 * _NC)), full((1, 4 * _NC)),
        ],
        out_specs=[
            full((n, _NC)),
            full((n, _NC)),
            full((n, 4 * _NC)),
        ],
        out_shape=[
            jax.ShapeDtypeStruct((n, _NC), jnp.float32),
            jax.ShapeDtypeStruct((n, _NC), jnp.float32),
            jax.ShapeDtypeStruct((n, 4 * _NC), jnp.float32),
        ],
        scratch_shapes=[
            pltpu.VMEM((n, _H), jnp.float32),
        ],
        compiler_params=pltpu.CompilerParams(
            dimension_semantics=("arbitrary",),
            vmem_limit_bytes=100 * 1024 * 1024,
        ),
    )(pooled_rois, conv1_w, row(conv1_b),
      row(bn1_gamma), row(bn1_beta), conv2_w, row(conv2_b), row(bn2_gamma),
      row(bn2_beta), logits_w, row(logits_b), delta_w, row(delta_b))
    return logits, probs, deltas.reshape(n, _NC, 4)
